# baseline (speedup 1.0000x reference)
import functools

import jax
import jax.numpy as jnp
from jax import lax
from jax.experimental import pallas as pl
from jax.experimental.pallas import tpu as pltpu
from jax.experimental.pallas import tpu_sc as plsc

NC = 2
NS = 16
LN = 16
NW = NC * NS
CK = 128
CHB = 16

F32 = jnp.float32


def _mesh():
    return plsc.VectorSubcoreMesh(core_axis_name="c", subcore_axis_name="s")


def _sc_degree(dst3, np_rows):
    _, CH, _ = dst3.shape
    rpt = np_rows // NS

    @functools.partial(
        pl.kernel,
        out_type=jax.ShapeDtypeStruct((NC, np_rows), F32),
        mesh=_mesh(),
        scratch_types=[
            pltpu.VMEM((CH, CK), jnp.int32),
            pltpu.VMEM((CK,), F32),
            pltpu.VMEM((rpt,), F32),
            pltpu.VMEM_SHARED((np_rows,), F32),
        ],
    )
    def k(dst_hbm, out_hbm, idx_d, ones_v, init_v, deg_sh):
        cid = lax.axis_index("c")
        sid = lax.axis_index("s")
        wid = cid * NS + sid
        base = sid * rpt
        iv = jnp.where(cid == 0, 1.0, 0.0).astype(F32)
        for i in range(CK // LN):
            ones_v[pl.ds(i * LN, LN)] = jnp.full((LN,), 1.0, F32)

        @pl.loop(0, rpt // LN)
        def _(i):
            init_v[pl.ds(i * LN, LN)] = jnp.zeros((LN,), F32) + iv

        pltpu.sync_copy(init_v, deg_sh.at[pl.ds(base, rpt)])
        pltpu.sync_copy(dst_hbm.at[wid], idx_d)
        plsc.subcore_barrier()

        @pl.loop(0, CH)
        def _(j):
            pltpu.sync_copy(ones_v, deg_sh.at[idx_d.at[j]], add=True)

        plsc.subcore_barrier()
        pltpu.sync_copy(deg_sh.at[pl.ds(base, rpt)],
                        out_hbm.at[cid, pl.ds(base, rpt)])

    return k(dst3)


def _sc_aggregate(hs, edges4):
    NP, D = hs.shape
    TOT_BLK = edges4.shape[0]
    NBP = TOT_BLK // NS
    CH = NBP * CHB
    rpt = NP // NS

    @functools.partial(
        pl.kernel,
        out_type=jax.ShapeDtypeStruct((NP, D), F32),
        mesh=_mesh(),
        scratch_types=[
            pltpu.VMEM((2, 2, CHB, CK), jnp.int32),
            pltpu.VMEM((2, CK, D), F32),
            pltpu.VMEM_SHARED((NP, D), F32),
            pltpu.SemaphoreType.DMA((2,)),
            pltpu.SemaphoreType.DMA,
        ],
    )
    def k(hs_hbm, e_hbm, out_hbm, iblk, rbuf, acc_sh, gsem, isem):
        cid = lax.axis_index("c")
        sid = lax.axis_index("s")
        blkbase = sid * NBP
        base = sid * rpt

        @pl.when(cid == 0)
        def _():
            pltpu.sync_copy(hs_hbm.at[pl.ds(base, rpt)],
                            acc_sh.at[pl.ds(base, rpt)])
            pltpu.sync_copy(e_hbm.at[blkbase], iblk.at[0])
            plsc.subcore_barrier()

            pltpu.async_copy(hs_hbm.at[iblk.at[0, 0, 0]], rbuf.at[0], gsem.at[0])

            @pl.loop(0, CH)
            def _(j):
                blk = j // CHB
                off = j % CHB
                ib = blk % 2
                b = j % 2
                @pl.when(jnp.logical_and(off == 0, blk + 1 < NBP))
                def _():
                    pltpu.async_copy(e_hbm.at[blkbase + blk + 1],
                                     iblk.at[1 - ib], isem)
                @pl.when(jnp.logical_and(off == CHB - 1, blk + 1 < NBP))
                def _():
                    pltpu.make_async_copy(e_hbm.at[blkbase + blk + 1],
                                          iblk.at[1 - ib], isem).wait()
                pltpu.make_async_copy(hs_hbm.at[iblk.at[ib, 0, off]],
                                      rbuf.at[b], gsem.at[b]).wait()
                @pl.when(j + 1 < CH)
                def _():
                    jn = j + 1
                    pltpu.async_copy(
                        hs_hbm.at[iblk.at[(jn // CHB) % 2, 0, jn % CHB]],
                        rbuf.at[1 - b], gsem.at[1 - b])
                pltpu.sync_copy(rbuf.at[b], acc_sh.at[iblk.at[ib, 1, off]],
                                add=True)

            plsc.subcore_barrier()
            pltpu.sync_copy(acc_sh.at[pl.ds(base, rpt)],
                            out_hbm.at[pl.ds(base, rpt)])

    return k(hs, edges4)


def _sc_cs_scatter(h2, u3, v3, zeros_hbm):
    NP, D = h2.shape
    _, CHM, _ = u3.shape
    rpt = NP // NS

    @functools.partial(
        pl.kernel,
        out_type=(jax.ShapeDtypeStruct((NP, D), F32),
                  jax.ShapeDtypeStruct((NP,), F32)),
        mesh=_mesh(),
        scratch_types=[
            pltpu.VMEM((CHM, CK), jnp.int32),
            pltpu.VMEM((CHM, CK), jnp.int32),
            pltpu.VMEM((2, CK, D), F32),
            pltpu.VMEM((CK,), F32),
            pltpu.VMEM((rpt,), F32),
            pltpu.VMEM_SHARED((NP, D), F32),
            pltpu.VMEM_SHARED((NP,), F32),
            pltpu.SemaphoreType.DMA((2,)),
            pltpu.SemaphoreType.DMA,
        ],
    )
    def k(h_hbm, u_hbm, v_hbm, z_hbm, t_hbm, c_hbm,
          idx_u, idx_v, rbuf, ones_v, zv, t_sh, c_sh, gsem, csem):
        cid = lax.axis_index("c")
        sid = lax.axis_index("s")
        base = sid * rpt

        @pl.when(cid == 0)
        def _():
            for i in range(CK // LN):
                ones_v[pl.ds(i * LN, LN)] = jnp.full((LN,), 1.0, F32)

            @pl.loop(0, rpt // LN)
            def _(i):
                zv[pl.ds(i * LN, LN)] = jnp.zeros((LN,), F32)

            pltpu.sync_copy(z_hbm.at[pl.ds(base, rpt)], t_sh.at[pl.ds(base, rpt)])
            pltpu.sync_copy(zv, c_sh.at[pl.ds(base, rpt)])
            pltpu.sync_copy(u_hbm.at[sid], idx_u)
            pltpu.sync_copy(v_hbm.at[sid], idx_v)
            plsc.subcore_barrier()

            pltpu.async_copy(h_hbm.at[idx_u.at[0]], rbuf.at[0], gsem.at[0])

            @pl.loop(0, CHM)
            def _(j):
                b = j % 2
                pltpu.make_async_copy(h_hbm.at[idx_u.at[j]], rbuf.at[b],
                                      gsem.at[b]).wait()
                @pl.when(j + 1 < CHM)
                def _():
                    pltpu.async_copy(h_hbm.at[idx_u.at[j + 1]], rbuf.at[1 - b],
                                     gsem.at[1 - b])
                pltpu.async_copy(ones_v, c_sh.at[idx_v.at[j]], csem, add=True)
                pltpu.sync_copy(rbuf.at[b], t_sh.at[idx_v.at[j]], add=True)

            @pl.loop(0, CHM)
            def _(j):
                pltpu.make_async_copy(ones_v, c_sh.at[idx_v.at[j]], csem).wait()

            plsc.subcore_barrier()
            pltpu.sync_copy(t_sh.at[pl.ds(base, rpt)], t_hbm.at[pl.ds(base, rpt)])
            pltpu.sync_copy(c_sh.at[pl.ds(base, rpt)], c_hbm.at[pl.ds(base, rpt)])

    return k(h2, u3, v3, zeros_hbm)


def _tc_pre(x, W_mlp, b_mlp, W_g1, degs, BR=512):
    NP, D = x.shape

    def body(x_r, wm_r, bm_r, wg_r, deg_r, hs_r, dinv_r):
        deg = deg_r[0] + deg_r[1]
        dinv = lax.rsqrt(deg)
        h0 = jnp.dot(x_r[...], wm_r[...], preferred_element_type=F32) + bm_r[...]
        g = jnp.dot(h0, wg_r[...], preferred_element_type=F32)
        hs_r[...] = g * dinv
        dinv_r[...] = dinv

    return pl.pallas_call(
        body,
        grid=(NP // BR,),
        in_specs=[
            pl.BlockSpec((BR, D), lambda i: (i, 0)),
            pl.BlockSpec((D, D), lambda i: (0, 0)),
            pl.BlockSpec((1, D), lambda i: (0, 0)),
            pl.BlockSpec((D, D), lambda i: (0, 0)),
            pl.BlockSpec((NC, BR, 1), lambda i: (0, i, 0)),
        ],
        out_specs=[pl.BlockSpec((BR, D), lambda i: (i, 0)),
                   pl.BlockSpec((BR, 1), lambda i: (i, 0))],
        out_shape=[jax.ShapeDtypeStruct((NP, D), F32),
                   jax.ShapeDtypeStruct((NP, 1), F32)],
    )(x, W_mlp, b_mlp.reshape(1, D), W_g1, degs[..., None])


def _tc_mid(p, dinv, b, W_next, BR=512):
    NP, D = p.shape

    def body(p_r, dinv_r, b_r, w_r, out_r):
        h = p_r[...] * dinv_r[...] + b_r[...]
        h = jnp.where(h > 0, h, jnp.exp(h) - 1.0)
        out_r[...] = jnp.dot(h, w_r[...], preferred_element_type=F32) * dinv_r[...]

    return pl.pallas_call(
        body,
        grid=(NP // BR,),
        in_specs=[
            pl.BlockSpec((BR, D), lambda i: (i, 0)),
            pl.BlockSpec((BR, 1), lambda i: (i, 0)),
            pl.BlockSpec((1, D), lambda i: (0, 0)),
            pl.BlockSpec((D, D), lambda i: (0, 0)),
        ],
        out_specs=pl.BlockSpec((BR, D), lambda i: (i, 0)),
        out_shape=jax.ShapeDtypeStruct((NP, D), F32),
    )(p, dinv, b.reshape(1, D), W_next)


def _tc_final(p, dinv, b, BR=512):
    NP, D = p.shape

    def body(p_r, dinv_r, b_r, out_r):
        h = p_r[...] * dinv_r[...] + b_r[...]
        out_r[...] = jnp.where(h > 0, h, jnp.exp(h) - 1.0)

    return pl.pallas_call(
        body,
        grid=(NP // BR,),
        in_specs=[
            pl.BlockSpec((BR, D), lambda i: (i, 0)),
            pl.BlockSpec((BR, 1), lambda i: (i, 0)),
            pl.BlockSpec((1, D), lambda i: (0, 0)),
        ],
        out_specs=pl.BlockSpec((BR, D), lambda i: (i, 0)),
        out_shape=jax.ShapeDtypeStruct((NP, D), F32),
    )(p, dinv, b.reshape(1, D))


def _tc_mean(t, c, BR=512):
    NP, D = t.shape

    def body(t_r, c_r, out_r):
        out_r[...] = t_r[...] / (1.0 + c_r[...])

    return pl.pallas_call(
        body,
        grid=(NP // BR,),
        in_specs=[
            pl.BlockSpec((BR, D), lambda i: (i, 0)),
            pl.BlockSpec((BR, 1), lambda i: (i, 0)),
        ],
        out_specs=pl.BlockSpec((BR, D), lambda i: (i, 0)),
        out_shape=jax.ShapeDtypeStruct((NP, D), F32),
    )(t, c[..., None])


def _pad_chunks(idx, fill, workers, n_chunks):
    total = workers * n_chunks * CK
    pad = total - idx.shape[0]
    idx = jnp.concatenate([idx, jnp.full((pad,), fill, jnp.int32)])
    return idx.reshape(workers, n_chunks, CK)


def kernel(x_q, x_t, edge_index_q, cs_u, cs_v,
           W_mlp, b_mlp, W_g1, b_g1, W_g2, b_g2):
    N, D = x_q.shape
    NT = x_t.shape[0]
    E = edge_index_q.shape[1]
    M = cs_u.shape[0]

    NP = ((max(N, NT) + 1 + NS * LN - 1) // (NS * LN)) * (NS * LN)
    TRASH = max(N, NT)

    NBP = (E + NS * CHB * CK - 1) // (NS * CHB * CK)
    NB0 = max(1, min(NBP - 1, (NBP * 8 + 5) // 10))
    NB1 = NBP - NB0
    TOT_BLK = NS * NBP
    CH = NBP * CHB
    CHM = (M + NS * CK - 1) // (NS * CK)

    src3 = _pad_chunks(edge_index_q[0], 0, NS, CH)
    dst3 = _pad_chunks(edge_index_q[1], TRASH, NS, CH)
    edges4 = jnp.stack([src3.reshape(TOT_BLK, CHB, CK),
                        dst3.reshape(TOT_BLK, CHB, CK)], axis=1)
    dst3 = dst3.reshape(NW, CH // 2, CK)
    u3 = _pad_chunks(cs_u, 0, NS, CHM)
    v3 = _pad_chunks(cs_v, TRASH, NS, CHM)

    x_pad = jnp.concatenate([x_q, jnp.zeros((NP - N, D), F32)])
    zeros_hbm = jnp.zeros((NP, D), F32)

    degs = _sc_degree(dst3, NP)
    hs1, dinv = _tc_pre(x_pad, W_mlp, b_mlp, W_g1, degs)
    p1 = _sc_aggregate(hs1, edges4)
    hs2 = _tc_mid(p1, dinv, b_g1, W_g2)
    p2 = _sc_aggregate(hs2, edges4)
    h2 = _tc_final(p2, dinv, b_g2)
    t, c = _sc_cs_scatter(h2, u3, v3, zeros_hbm)
    xt = _tc_mean(t, c)

    return h2[:N], xt[:NT]


# speedup vs baseline: 2.8189x; 2.8189x over previous
import functools

import jax
import jax.numpy as jnp
from jax import lax
from jax.experimental import pallas as pl
from jax.experimental.pallas import tpu as pltpu
from jax.experimental.pallas import tpu_sc as plsc

NC = 2
NS = 16
LN = 16
NW = NC * NS
CK = 128
CHB = 16

F32 = jnp.float32


def _mesh():
    return plsc.VectorSubcoreMesh(core_axis_name="c", subcore_axis_name="s")


def _sc_degree(dst3, np_rows):
    _, CH, _ = dst3.shape
    rpt = np_rows // NS

    @functools.partial(
        pl.kernel,
        out_type=jax.ShapeDtypeStruct((NC, np_rows), F32),
        mesh=_mesh(),
        scratch_types=[
            pltpu.VMEM((CH, CK), jnp.int32),
            pltpu.VMEM((CK,), F32),
            pltpu.VMEM((rpt,), F32),
            pltpu.VMEM_SHARED((np_rows,), F32),
        ],
    )
    def k(dst_hbm, out_hbm, idx_d, ones_v, init_v, deg_sh):
        cid = lax.axis_index("c")
        sid = lax.axis_index("s")
        wid = cid * NS + sid
        base = sid * rpt
        iv = jnp.where(cid == 0, 1.0, 0.0).astype(F32)
        for i in range(CK // LN):
            ones_v[pl.ds(i * LN, LN)] = jnp.full((LN,), 1.0, F32)

        @pl.loop(0, rpt // LN)
        def _(i):
            init_v[pl.ds(i * LN, LN)] = jnp.zeros((LN,), F32) + iv

        pltpu.sync_copy(init_v, deg_sh.at[pl.ds(base, rpt)])
        pltpu.sync_copy(dst_hbm.at[wid], idx_d)
        plsc.subcore_barrier()

        @pl.loop(0, CH)
        def _(j):
            pltpu.sync_copy(ones_v, deg_sh.at[idx_d.at[j]], add=True)

        plsc.subcore_barrier()
        pltpu.sync_copy(deg_sh.at[pl.ds(base, rpt)],
                        out_hbm.at[cid, pl.ds(base, rpt)])

    return k(dst3)


def _sc_aggregate(hs, edges4, zeros_hbm):
    NP, D = hs.shape
    TOT_BLK = edges4.shape[0]
    NBW = TOT_BLK // NW
    CH = NBW * CHB
    rpt = NP // NS

    @functools.partial(
        pl.kernel,
        out_type=jax.ShapeDtypeStruct((NC, NP, D), F32),
        mesh=_mesh(),
        scratch_types=[
            pltpu.VMEM((2, 2, CHB, CK), jnp.int32),
            pltpu.VMEM((2, CK, D), F32),
            pltpu.VMEM_SHARED((NP, D), F32),
            pltpu.SemaphoreType.DMA((2,)),
            pltpu.SemaphoreType.DMA,
        ],
    )
    def k(hs_hbm, e_hbm, z_hbm, out_hbm, iblk, rbuf, acc_sh, gsem, isem):
        cid = lax.axis_index("c")
        sid = lax.axis_index("s")
        blkbase = (cid * NS + sid) * NBW
        base = sid * rpt

        @pl.when(cid == 0)
        def _():
            pltpu.sync_copy(hs_hbm.at[pl.ds(base, rpt)],
                            acc_sh.at[pl.ds(base, rpt)])
        @pl.when(cid == 1)
        def _():
            pltpu.sync_copy(z_hbm.at[pl.ds(base, rpt)],
                            acc_sh.at[pl.ds(base, rpt)])
        pltpu.sync_copy(e_hbm.at[blkbase], iblk.at[0])
        plsc.subcore_barrier()

        pltpu.async_copy(hs_hbm.at[iblk.at[0, 0, 0]], rbuf.at[0], gsem.at[0])

        @pl.loop(0, CH)
        def _(j):
            blk = j // CHB
            off = j % CHB
            ib = blk % 2
            b = j % 2
            @pl.when(jnp.logical_and(off == 0, blk + 1 < NBW))
            def _():
                pltpu.async_copy(e_hbm.at[blkbase + blk + 1],
                                 iblk.at[1 - ib], isem)
            @pl.when(jnp.logical_and(off == CHB - 1, blk + 1 < NBW))
            def _():
                pltpu.make_async_copy(e_hbm.at[blkbase + blk + 1],
                                      iblk.at[1 - ib], isem).wait()
            pltpu.make_async_copy(hs_hbm.at[iblk.at[ib, 0, off]],
                                  rbuf.at[b], gsem.at[b]).wait()
            @pl.when(j + 1 < CH)
            def _():
                jn = j + 1
                pltpu.async_copy(
                    hs_hbm.at[iblk.at[(jn // CHB) % 2, 0, jn % CHB]],
                    rbuf.at[1 - b], gsem.at[1 - b])
            pltpu.sync_copy(rbuf.at[b], acc_sh.at[iblk.at[ib, 1, off]],
                            add=True)

        plsc.subcore_barrier()
        pltpu.sync_copy(acc_sh.at[pl.ds(base, rpt)],
                        out_hbm.at[cid, pl.ds(base, rpt)])

    return k(hs, edges4, zeros_hbm)


def _sc_cs_scatter(h2, u3, v3, zeros_hbm):
    NP, D = h2.shape
    _, CHM, _ = u3.shape
    rpt = NP // NS

    @functools.partial(
        pl.kernel,
        out_type=(jax.ShapeDtypeStruct((NP, D), F32),
                  jax.ShapeDtypeStruct((NP,), F32)),
        mesh=_mesh(),
        scratch_types=[
            pltpu.VMEM((CHM, CK), jnp.int32),
            pltpu.VMEM((CHM, CK), jnp.int32),
            pltpu.VMEM((2, CK, D), F32),
            pltpu.VMEM((CK,), F32),
            pltpu.VMEM((rpt,), F32),
            pltpu.VMEM_SHARED((NP, D), F32),
            pltpu.VMEM_SHARED((NP,), F32),
            pltpu.SemaphoreType.DMA((2,)),
            pltpu.SemaphoreType.DMA,
        ],
    )
    def k(h_hbm, u_hbm, v_hbm, z_hbm, t_hbm, c_hbm,
          idx_u, idx_v, rbuf, ones_v, zv, t_sh, c_sh, gsem, csem):
        cid = lax.axis_index("c")
        sid = lax.axis_index("s")
        base = sid * rpt

        @pl.when(cid == 0)
        def _():
            for i in range(CK // LN):
                ones_v[pl.ds(i * LN, LN)] = jnp.full((LN,), 1.0, F32)

            @pl.loop(0, rpt // LN)
            def _(i):
                zv[pl.ds(i * LN, LN)] = jnp.zeros((LN,), F32)

            pltpu.sync_copy(z_hbm.at[pl.ds(base, rpt)], t_sh.at[pl.ds(base, rpt)])
            pltpu.sync_copy(zv, c_sh.at[pl.ds(base, rpt)])
            pltpu.sync_copy(u_hbm.at[sid], idx_u)
            pltpu.sync_copy(v_hbm.at[sid], idx_v)
            plsc.subcore_barrier()

            pltpu.async_copy(h_hbm.at[idx_u.at[0]], rbuf.at[0], gsem.at[0])

            @pl.loop(0, CHM)
            def _(j):
                b = j % 2
                pltpu.make_async_copy(h_hbm.at[idx_u.at[j]], rbuf.at[b],
                                      gsem.at[b]).wait()
                @pl.when(j + 1 < CHM)
                def _():
                    pltpu.async_copy(h_hbm.at[idx_u.at[j + 1]], rbuf.at[1 - b],
                                     gsem.at[1 - b])
                pltpu.async_copy(ones_v, c_sh.at[idx_v.at[j]], csem, add=True)
                pltpu.sync_copy(rbuf.at[b], t_sh.at[idx_v.at[j]], add=True)

            @pl.loop(0, CHM)
            def _(j):
                pltpu.make_async_copy(ones_v, c_sh.at[idx_v.at[j]], csem).wait()

            plsc.subcore_barrier()
            pltpu.sync_copy(t_sh.at[pl.ds(base, rpt)], t_hbm.at[pl.ds(base, rpt)])
            pltpu.sync_copy(c_sh.at[pl.ds(base, rpt)], c_hbm.at[pl.ds(base, rpt)])

    return k(h2, u3, v3, zeros_hbm)


def _tc_pre(x, W_mlp, b_mlp, W_g1, degs, BR=512):
    NP, D = x.shape

    def body(x_r, wm_r, bm_r, wg_r, deg_r, hs_r, dinv_r):
        deg = deg_r[0] + deg_r[1]
        dinv = lax.rsqrt(deg)
        h0 = jnp.dot(x_r[...], wm_r[...], preferred_element_type=F32) + bm_r[...]
        g = jnp.dot(h0, wg_r[...], preferred_element_type=F32)
        hs_r[...] = g * dinv
        dinv_r[...] = dinv

    return pl.pallas_call(
        body,
        grid=(NP // BR,),
        in_specs=[
            pl.BlockSpec((BR, D), lambda i: (i, 0)),
            pl.BlockSpec((D, D), lambda i: (0, 0)),
            pl.BlockSpec((1, D), lambda i: (0, 0)),
            pl.BlockSpec((D, D), lambda i: (0, 0)),
            pl.BlockSpec((NC, BR, 1), lambda i: (0, i, 0)),
        ],
        out_specs=[pl.BlockSpec((BR, D), lambda i: (i, 0)),
                   pl.BlockSpec((BR, 1), lambda i: (i, 0))],
        out_shape=[jax.ShapeDtypeStruct((NP, D), F32),
                   jax.ShapeDtypeStruct((NP, 1), F32)],
    )(x, W_mlp, b_mlp.reshape(1, D), W_g1, degs[..., None])


def _tc_mid(p, dinv, b, W_next, BR=512):
    _, NP, D = p.shape

    def body(p_r, dinv_r, b_r, w_r, out_r):
        h = (p_r[0] + p_r[1]) * dinv_r[...] + b_r[...]
        h = jnp.where(h > 0, h, jnp.exp(h) - 1.0)
        out_r[...] = jnp.dot(h, w_r[...], preferred_element_type=F32) * dinv_r[...]

    return pl.pallas_call(
        body,
        grid=(NP // BR,),
        in_specs=[
            pl.BlockSpec((NC, BR, D), lambda i: (0, i, 0)),
            pl.BlockSpec((BR, 1), lambda i: (i, 0)),
            pl.BlockSpec((1, D), lambda i: (0, 0)),
            pl.BlockSpec((D, D), lambda i: (0, 0)),
        ],
        out_specs=pl.BlockSpec((BR, D), lambda i: (i, 0)),
        out_shape=jax.ShapeDtypeStruct((NP, D), F32),
    )(p, dinv, b.reshape(1, D), W_next)


def _tc_final(p, dinv, b, BR=512):
    _, NP, D = p.shape

    def body(p_r, dinv_r, b_r, out_r):
        h = (p_r[0] + p_r[1]) * dinv_r[...] + b_r[...]
        out_r[...] = jnp.where(h > 0, h, jnp.exp(h) - 1.0)

    return pl.pallas_call(
        body,
        grid=(NP // BR,),
        in_specs=[
            pl.BlockSpec((NC, BR, D), lambda i: (0, i, 0)),
            pl.BlockSpec((BR, 1), lambda i: (i, 0)),
            pl.BlockSpec((1, D), lambda i: (0, 0)),
        ],
        out_specs=pl.BlockSpec((BR, D), lambda i: (i, 0)),
        out_shape=jax.ShapeDtypeStruct((NP, D), F32),
    )(p, dinv, b.reshape(1, D))


def _tc_mean(t, c, BR=512):
    NP, D = t.shape

    def body(t_r, c_r, out_r):
        out_r[...] = t_r[...] / (1.0 + c_r[...])

    return pl.pallas_call(
        body,
        grid=(NP // BR,),
        in_specs=[
            pl.BlockSpec((BR, D), lambda i: (i, 0)),
            pl.BlockSpec((BR, 1), lambda i: (i, 0)),
        ],
        out_specs=pl.BlockSpec((BR, D), lambda i: (i, 0)),
        out_shape=jax.ShapeDtypeStruct((NP, D), F32),
    )(t, c[..., None])


def _pad_chunks(idx, fill_lo, fill_n, workers, n_chunks):
    total = workers * n_chunks * CK
    pad = total - idx.shape[0]
    padv = fill_lo + jnp.arange(pad, dtype=jnp.int32) % fill_n
    return jnp.concatenate([idx, padv]).reshape(workers, n_chunks, CK)


def kernel(x_q, x_t, edge_index_q, cs_u, cs_v,
           W_mlp, b_mlp, W_g1, b_g1, W_g2, b_g2):
    N, D = x_q.shape
    NT = x_t.shape[0]
    E = edge_index_q.shape[1]
    M = cs_u.shape[0]

    NP = ((max(N, NT) + 1 + NS * LN - 1) // (NS * LN)) * (NS * LN)
    TRASH = max(N, NT)

    NBP = (E + NS * CHB * CK - 1) // (NS * CHB * CK)
    NB0 = max(1, min(NBP - 1, (NBP * 8 + 5) // 10))
    NB1 = NBP - NB0
    TOT_BLK = NS * NBP
    CH = NBP * CHB
    CHM = (M + NS * CK - 1) // (NS * CK)

    src3 = _pad_chunks(edge_index_q[0], 0, N, NS, CH)
    dst3 = _pad_chunks(edge_index_q[1], TRASH, NP - TRASH, NS, CH)
    edges4 = jnp.stack([src3.reshape(TOT_BLK, CHB, CK),
                        dst3.reshape(TOT_BLK, CHB, CK)], axis=1)
    dst3 = dst3.reshape(NW, CH // 2, CK)
    u3 = _pad_chunks(cs_u, 0, N, NS, CHM)
    v3 = _pad_chunks(cs_v, TRASH, NP - TRASH, NS, CHM)

    x_pad = jnp.concatenate([x_q, jnp.zeros((NP - N, D), F32)])
    zeros_hbm = jnp.zeros((NP, D), F32)

    degs = _sc_degree(dst3, NP)
    hs1, dinv = _tc_pre(x_pad, W_mlp, b_mlp, W_g1, degs)
    p1 = _sc_aggregate(hs1, edges4, zeros_hbm)
    hs2 = _tc_mid(p1, dinv, b_g1, W_g2)
    p2 = _sc_aggregate(hs2, edges4, zeros_hbm)
    h2 = _tc_final(p2, dinv, b_g2)
    t, c = _sc_cs_scatter(h2, u3, v3, zeros_hbm)
    xt = _tc_mean(t, c)

    return h2[:N], xt[:NT]


# speedup vs baseline: 2.8505x; 1.0112x over previous
import functools

import jax
import jax.numpy as jnp
from jax import lax
from jax.experimental import pallas as pl
from jax.experimental.pallas import tpu as pltpu
from jax.experimental.pallas import tpu_sc as plsc

NC = 2
NS = 16
LN = 16
NW = NC * NS
CK = 128
CHB = 16

F32 = jnp.float32


def _mesh():
    return plsc.VectorSubcoreMesh(core_axis_name="c", subcore_axis_name="s")


def _sc_degree(dst3, np_rows):
    _, CH, _ = dst3.shape
    rpt = np_rows // NS

    @functools.partial(
        pl.kernel,
        out_type=jax.ShapeDtypeStruct((NC, np_rows), F32),
        mesh=_mesh(),
        scratch_types=[
            pltpu.VMEM((CH, CK), jnp.int32),
            pltpu.VMEM((CK,), F32),
            pltpu.VMEM((rpt,), F32),
            pltpu.VMEM_SHARED((np_rows,), F32),
        ],
    )
    def k(dst_hbm, out_hbm, idx_d, ones_v, init_v, deg_sh):
        cid = lax.axis_index("c")
        sid = lax.axis_index("s")
        wid = cid * NS + sid
        base = sid * rpt
        iv = jnp.where(cid == 0, 1.0, 0.0).astype(F32)
        for i in range(CK // LN):
            ones_v[pl.ds(i * LN, LN)] = jnp.full((LN,), 1.0, F32)

        @pl.loop(0, rpt // LN)
        def _(i):
            init_v[pl.ds(i * LN, LN)] = jnp.zeros((LN,), F32) + iv

        pltpu.sync_copy(init_v, deg_sh.at[pl.ds(base, rpt)])
        pltpu.sync_copy(dst_hbm.at[wid], idx_d)
        plsc.subcore_barrier()

        @pl.loop(0, CH)
        def _(j):
            pltpu.sync_copy(ones_v, deg_sh.at[idx_d.at[j]], add=True)

        plsc.subcore_barrier()
        pltpu.sync_copy(deg_sh.at[pl.ds(base, rpt)],
                        out_hbm.at[cid, pl.ds(base, rpt)])

    return k(dst3)


def _sc_aggregate(hs, edges4, zeros_hbm):
    NP, D = hs.shape
    TOT_BLK = edges4.shape[0]
    NBW = TOT_BLK // NW
    CH = NBW * CHB
    rpt = NP // NS

    @functools.partial(
        pl.kernel,
        out_type=jax.ShapeDtypeStruct((NC, NP, D), F32),
        mesh=_mesh(),
        scratch_types=[
            pltpu.VMEM((2, 2, CHB, CK), jnp.int32),
            pltpu.VMEM((2, CK, D), F32),
            pltpu.VMEM_SHARED((NP, D), F32),
            pltpu.SemaphoreType.DMA((2,)),
            pltpu.SemaphoreType.DMA,
        ],
    )
    def k(hs_hbm, e_hbm, z_hbm, out_hbm, iblk, rbuf, acc_sh, gsem, isem):
        cid = lax.axis_index("c")
        sid = lax.axis_index("s")
        blkbase = (cid * NS + sid) * NBW
        base = sid * rpt

        @pl.when(cid == 0)
        def _():
            pltpu.sync_copy(hs_hbm.at[pl.ds(base, rpt)],
                            acc_sh.at[pl.ds(base, rpt)])
        @pl.when(cid == 1)
        def _():
            pltpu.sync_copy(z_hbm.at[pl.ds(base, rpt)],
                            acc_sh.at[pl.ds(base, rpt)])
        pltpu.sync_copy(e_hbm.at[blkbase], iblk.at[0])
        plsc.subcore_barrier()

        pltpu.async_copy(hs_hbm.at[iblk.at[0, 0, 0]], rbuf.at[0], gsem.at[0])

        @pl.loop(0, CH)
        def _(j):
            blk = j // CHB
            off = j % CHB
            ib = blk % 2
            b = j % 2
            @pl.when(jnp.logical_and(off == 0, blk + 1 < NBW))
            def _():
                pltpu.async_copy(e_hbm.at[blkbase + blk + 1],
                                 iblk.at[1 - ib], isem)
            @pl.when(jnp.logical_and(off == CHB - 1, blk + 1 < NBW))
            def _():
                pltpu.make_async_copy(e_hbm.at[blkbase + blk + 1],
                                      iblk.at[1 - ib], isem).wait()
            pltpu.make_async_copy(hs_hbm.at[iblk.at[ib, 0, off]],
                                  rbuf.at[b], gsem.at[b]).wait()
            @pl.when(j + 1 < CH)
            def _():
                jn = j + 1
                pltpu.async_copy(
                    hs_hbm.at[iblk.at[(jn // CHB) % 2, 0, jn % CHB]],
                    rbuf.at[1 - b], gsem.at[1 - b])
            pltpu.sync_copy(rbuf.at[b], acc_sh.at[iblk.at[ib, 1, off]],
                            add=True)

        plsc.subcore_barrier()
        pltpu.sync_copy(acc_sh.at[pl.ds(base, rpt)],
                        out_hbm.at[cid, pl.ds(base, rpt)])

    return k(hs, edges4, zeros_hbm)


def _sc_cs_scatter(h2, u3, v3, zeros_hbm, np_rows):
    _, D = h2.shape
    NP = np_rows
    _, CHM, _ = u3.shape
    rpt = NP // NS

    @functools.partial(
        pl.kernel,
        out_type=(jax.ShapeDtypeStruct((NP, D), F32),
                  jax.ShapeDtypeStruct((NP,), F32)),
        mesh=_mesh(),
        scratch_types=[
            pltpu.VMEM((CHM, CK), jnp.int32),
            pltpu.VMEM((CHM, CK), jnp.int32),
            pltpu.VMEM((2, CK, D), F32),
            pltpu.VMEM((CK,), F32),
            pltpu.VMEM((rpt,), F32),
            pltpu.VMEM_SHARED((NP, D), F32),
            pltpu.VMEM_SHARED((NP,), F32),
            pltpu.SemaphoreType.DMA((2,)),
            pltpu.SemaphoreType.DMA,
        ],
    )
    def k(h_hbm, u_hbm, v_hbm, z_hbm, t_hbm, c_hbm,
          idx_u, idx_v, rbuf, ones_v, zv, t_sh, c_sh, gsem, csem):
        cid = lax.axis_index("c")
        sid = lax.axis_index("s")
        base = sid * rpt

        @pl.when(cid == 0)
        def _():
            for i in range(CK // LN):
                ones_v[pl.ds(i * LN, LN)] = jnp.full((LN,), 1.0, F32)

            @pl.loop(0, rpt // LN)
            def _(i):
                zv[pl.ds(i * LN, LN)] = jnp.zeros((LN,), F32)

            pltpu.sync_copy(z_hbm.at[pl.ds(base, rpt)], t_sh.at[pl.ds(base, rpt)])
            pltpu.sync_copy(zv, c_sh.at[pl.ds(base, rpt)])
            pltpu.sync_copy(u_hbm.at[sid], idx_u)
            pltpu.sync_copy(v_hbm.at[sid], idx_v)
            plsc.subcore_barrier()

            pltpu.async_copy(h_hbm.at[idx_u.at[0]], rbuf.at[0], gsem.at[0])

            @pl.loop(0, CHM)
            def _(j):
                b = j % 2
                pltpu.make_async_copy(h_hbm.at[idx_u.at[j]], rbuf.at[b],
                                      gsem.at[b]).wait()
                @pl.when(j + 1 < CHM)
                def _():
                    pltpu.async_copy(h_hbm.at[idx_u.at[j + 1]], rbuf.at[1 - b],
                                     gsem.at[1 - b])
                pltpu.async_copy(ones_v, c_sh.at[idx_v.at[j]], csem, add=True)
                pltpu.sync_copy(rbuf.at[b], t_sh.at[idx_v.at[j]], add=True)

            @pl.loop(0, CHM)
            def _(j):
                pltpu.make_async_copy(ones_v, c_sh.at[idx_v.at[j]], csem).wait()

            plsc.subcore_barrier()
            pltpu.sync_copy(t_sh.at[pl.ds(base, rpt)], t_hbm.at[pl.ds(base, rpt)])
            pltpu.sync_copy(c_sh.at[pl.ds(base, rpt)], c_hbm.at[pl.ds(base, rpt)])

    return k(h2, u3, v3, zeros_hbm)


def _tc_pre(x, W_mlp, b_mlp, W_g1, degs, BR=512):
    NP, D = x.shape

    def body(x_r, wm_r, bm_r, wg_r, deg_r, hs_r, dinv_r):
        deg = deg_r[0] + deg_r[1]
        dinv = lax.rsqrt(deg)
        h0 = jnp.dot(x_r[...], wm_r[...], preferred_element_type=F32) + bm_r[...]
        g = jnp.dot(h0, wg_r[...], preferred_element_type=F32)
        hs_r[...] = g * dinv
        dinv_r[...] = dinv

    return pl.pallas_call(
        body,
        grid=(NP // BR,),
        in_specs=[
            pl.BlockSpec((BR, D), lambda i: (i, 0)),
            pl.BlockSpec((D, D), lambda i: (0, 0)),
            pl.BlockSpec((1, D), lambda i: (0, 0)),
            pl.BlockSpec((D, D), lambda i: (0, 0)),
            pl.BlockSpec((NC, BR, 1), lambda i: (0, i, 0)),
        ],
        out_specs=[pl.BlockSpec((BR, D), lambda i: (i, 0)),
                   pl.BlockSpec((BR, 1), lambda i: (i, 0))],
        out_shape=[jax.ShapeDtypeStruct((NP, D), F32),
                   jax.ShapeDtypeStruct((NP, 1), F32)],
    )(x, W_mlp, b_mlp.reshape(1, D), W_g1, degs[..., None])


def _tc_mid(p, dinv, b, W_next, BR=512):
    _, NP, D = p.shape

    def body(p_r, dinv_r, b_r, w_r, out_r):
        h = (p_r[0] + p_r[1]) * dinv_r[...] + b_r[...]
        h = jnp.where(h > 0, h, jnp.exp(h) - 1.0)
        out_r[...] = jnp.dot(h, w_r[...], preferred_element_type=F32) * dinv_r[...]

    return pl.pallas_call(
        body,
        grid=(NP // BR,),
        in_specs=[
            pl.BlockSpec((NC, BR, D), lambda i: (0, i, 0)),
            pl.BlockSpec((BR, 1), lambda i: (i, 0)),
            pl.BlockSpec((1, D), lambda i: (0, 0)),
            pl.BlockSpec((D, D), lambda i: (0, 0)),
        ],
        out_specs=pl.BlockSpec((BR, D), lambda i: (i, 0)),
        out_shape=jax.ShapeDtypeStruct((NP, D), F32),
    )(p, dinv, b.reshape(1, D), W_next)


def _tc_final(p, dinv, b, nrows, BR=512):
    _, NP, D = p.shape

    def body(p_r, dinv_r, b_r, out_r):
        h = (p_r[0] + p_r[1]) * dinv_r[...] + b_r[...]
        out_r[...] = jnp.where(h > 0, h, jnp.exp(h) - 1.0)

    return pl.pallas_call(
        body,
        grid=(NP // BR,),
        in_specs=[
            pl.BlockSpec((NC, BR, D), lambda i: (0, i, 0)),
            pl.BlockSpec((BR, 1), lambda i: (i, 0)),
            pl.BlockSpec((1, D), lambda i: (0, 0)),
        ],
        out_specs=pl.BlockSpec((BR, D), lambda i: (i, 0)),
        out_shape=jax.ShapeDtypeStruct((nrows, D), F32),
    )(p, dinv, b.reshape(1, D))


def _tc_mean(t, c, nrows, BR=512):
    NP, D = t.shape

    def body(t_r, c_r, out_r):
        out_r[...] = t_r[...] / (1.0 + c_r[...])

    return pl.pallas_call(
        body,
        grid=(NP // BR,),
        in_specs=[
            pl.BlockSpec((BR, D), lambda i: (i, 0)),
            pl.BlockSpec((BR, 1), lambda i: (i, 0)),
        ],
        out_specs=pl.BlockSpec((BR, D), lambda i: (i, 0)),
        out_shape=jax.ShapeDtypeStruct((nrows, D), F32),
    )(t, c[..., None])


def _pad_chunks(idx, fill_lo, fill_n, workers, n_chunks):
    total = workers * n_chunks * CK
    pad = total - idx.shape[0]
    padv = fill_lo + jnp.arange(pad, dtype=jnp.int32) % fill_n
    return jnp.concatenate([idx, padv]).reshape(workers, n_chunks, CK)


def kernel(x_q, x_t, edge_index_q, cs_u, cs_v,
           W_mlp, b_mlp, W_g1, b_g1, W_g2, b_g2):
    N, D = x_q.shape
    NT = x_t.shape[0]
    E = edge_index_q.shape[1]
    M = cs_u.shape[0]

    NP = ((max(N, NT) + 1 + NS * LN - 1) // (NS * LN)) * (NS * LN)
    TRASH = max(N, NT)

    NBP = (E + NS * CHB * CK - 1) // (NS * CHB * CK)
    NB0 = max(1, min(NBP - 1, (NBP * 8 + 5) // 10))
    NB1 = NBP - NB0
    TOT_BLK = NS * NBP
    CH = NBP * CHB
    CHM = (M + NS * CK - 1) // (NS * CK)

    src3 = _pad_chunks(edge_index_q[0], 0, N, NS, CH)
    dst3 = _pad_chunks(edge_index_q[1], TRASH, NP - TRASH, NS, CH)
    edges4 = jnp.stack([src3.reshape(TOT_BLK, CHB, CK),
                        dst3.reshape(TOT_BLK, CHB, CK)], axis=1)
    dst3 = dst3.reshape(NW, CH // 2, CK)
    u3 = _pad_chunks(cs_u, 0, N, NS, CHM)
    v3 = _pad_chunks(cs_v, TRASH, NP - TRASH, NS, CHM)

    x_pad = jnp.concatenate([x_q, jnp.zeros((NP - N, D), F32)])
    zeros_hbm = jnp.zeros((NP, D), F32)

    degs = _sc_degree(dst3, NP)
    hs1, dinv = _tc_pre(x_pad, W_mlp, b_mlp, W_g1, degs)
    p1 = _sc_aggregate(hs1, edges4, zeros_hbm)
    hs2 = _tc_mid(p1, dinv, b_g1, W_g2)
    p2 = _sc_aggregate(hs2, edges4, zeros_hbm)
    h2 = _tc_final(p2, dinv, b_g2, N)
    t, c = _sc_cs_scatter(h2, u3, v3, zeros_hbm, NP)
    xt = _tc_mean(t, c, NT)

    return h2, xt
